# Initial kernel scaffold; baseline (speedup 1.0000x reference)
import jax
import jax.numpy as jnp
from jax.experimental import pallas as pl


def kernel(x, graph_index, gamma1, beta1, W1, b1, W2, b2, gamma2, beta2):
    raise NotImplementedError("write your pallas kernel here")


# speedup vs baseline: 5.5783x; 5.5783x over previous
import functools

import jax
import jax.numpy as jnp
from jax import lax
from jax.experimental import pallas as pl
from jax.experimental.pallas import tpu as pltpu
from jax.experimental.pallas import tpu_sc as plsc

N = 320000
D = 128
S = 1024

NC = 2
NS = 16
NW = NC * NS
ROWS_PER_TILE = N // NW
CHUNK = 400
NCHUNKS = ROWS_PER_TILE // CHUNK
SUB = 80
NSUB = CHUNK // SUB
SEG_PER_TILE = S // NS


def _sc_segment_sum(x, idx, zsums, zcounts, ones):
    mesh = plsc.VectorSubcoreMesh(core_axis_name="c", subcore_axis_name="s")

    @functools.partial(
        pl.kernel,
        out_type=[
            jax.ShapeDtypeStruct((NC, S, D), jnp.float32),
            jax.ShapeDtypeStruct((NC * S,), jnp.float32),
        ],
        mesh=mesh,
        scratch_types=[
            pltpu.VMEM((CHUNK, D), jnp.float32),
            [pltpu.VMEM((SUB,), jnp.int32) for _ in range(NSUB)],
            pltpu.VMEM((SUB,), jnp.float32),
            pltpu.VMEM((SEG_PER_TILE,), jnp.float32),
            pltpu.VMEM_SHARED((S, D), jnp.float32),
            pltpu.VMEM_SHARED((S,), jnp.float32),
        ],
    )
    def seg_sum(x_hbm, idx_hbm, zs_hbm, zc_hbm, ones_hbm, out_sums, out_counts,
                xbuf, idxbufs, onesbuf, cbuf, acc_sums, acc_counts):
        cid = lax.axis_index("c")
        sid = lax.axis_index("s")
        wid = cid * NS + sid
        seg0 = sid * SEG_PER_TILE

        pltpu.sync_copy(zs_hbm.at[pl.ds(seg0, SEG_PER_TILE)],
                        acc_sums.at[pl.ds(seg0, SEG_PER_TILE)])
        pltpu.sync_copy(zc_hbm.at[pl.ds(seg0, SEG_PER_TILE)], cbuf)
        pltpu.sync_copy(cbuf, acc_counts.at[pl.ds(seg0, SEG_PER_TILE)])
        pltpu.sync_copy(ones_hbm, onesbuf)
        plsc.subcore_barrier()

        row0 = wid * ROWS_PER_TILE

        def body(ci, carry):
            base = row0 + ci * CHUNK
            pltpu.sync_copy(x_hbm.at[pl.ds(base, CHUNK)], xbuf)
            for j in range(NSUB):
                pltpu.sync_copy(idx_hbm.at[pl.ds(base + j * SUB, SUB)],
                                idxbufs[j])
            for j in range(NSUB):
                pltpu.sync_copy(xbuf.at[pl.ds(j * SUB, SUB)],
                                acc_sums.at[idxbufs[j]], add=True)
                pltpu.sync_copy(onesbuf, acc_counts.at[idxbufs[j]], add=True)
            return carry

        lax.fori_loop(0, NCHUNKS, body, 0)
        plsc.subcore_barrier()

        pltpu.sync_copy(acc_sums.at[pl.ds(seg0, SEG_PER_TILE)],
                        out_sums.at[cid, pl.ds(seg0, SEG_PER_TILE)])
        pltpu.sync_copy(acc_counts.at[pl.ds(seg0, SEG_PER_TILE)], cbuf)
        pltpu.sync_copy(cbuf, out_counts.at[pl.ds(cid * S + seg0, SEG_PER_TILE)])

    return seg_sum(x, idx, zsums, zcounts, ones)


def _finalize(psums, pcounts_t, gamma1, beta1, W1, b1, W2, b2, gamma2, beta2):
    def body(ps_ref, pc_ref, g1_ref, be1_ref, w1_ref, b1_ref, w2_ref, b2_ref,
             g2_ref, be2_ref, out_ref):
        sums = ps_ref[0] + ps_ref[1]
        counts = jnp.sum(pc_ref[...], axis=1, keepdims=True)
        counts = jnp.maximum(counts, 1.0)
        h = sums / counts

        mean = jnp.mean(h, axis=-1, keepdims=True)
        var = jnp.mean((h - mean) * (h - mean), axis=-1, keepdims=True)
        h = (h - mean) * lax.rsqrt(var + 1e-5) * g1_ref[...] + be1_ref[...]

        y = lax.dot_general(h, w1_ref[...], (((1,), (1,)), ((), ())),
                            preferred_element_type=jnp.float32) + b1_ref[...]
        y = jnp.where(y > 0.0, y, jnp.exp(jnp.minimum(y, 0.0)) - 1.0)
        y = lax.dot_general(y, w2_ref[...], (((1,), (1,)), ((), ())),
                            preferred_element_type=jnp.float32) + b2_ref[...]
        y = y + h

        mean2 = jnp.mean(y, axis=-1, keepdims=True)
        var2 = jnp.mean((y - mean2) * (y - mean2), axis=-1, keepdims=True)
        out_ref[...] = ((y - mean2) * lax.rsqrt(var2 + 1e-5) * g2_ref[...]
                        + be2_ref[...])

    return pl.pallas_call(
        body,
        out_shape=jax.ShapeDtypeStruct((S, D), jnp.float32),
    )(psums, pcounts_t, gamma1.reshape(1, D), beta1.reshape(1, D), W1,
      b1.reshape(1, D), W2, b2.reshape(1, D), gamma2.reshape(1, D),
      beta2.reshape(1, D))


def kernel(x, graph_index, gamma1, beta1, W1, b1, W2, b2, gamma2, beta2):
    idx = graph_index.astype(jnp.int32)
    zsums = jnp.zeros((S, D), jnp.float32)
    zcounts = jnp.zeros((S,), jnp.float32)
    ones = jnp.ones((SUB,), jnp.float32)
    psums, pcounts = _sc_segment_sum(x, idx, zsums, zcounts, ones)
    pcounts_t = pcounts.reshape(NC, S).T
    return _finalize(psums, pcounts_t, gamma1, beta1, W1, b1, W2, b2, gamma2,
                     beta2)


# speedup vs baseline: 7.3647x; 1.3203x over previous
import functools

import jax
import jax.numpy as jnp
from jax import lax
from jax.experimental import pallas as pl
from jax.experimental.pallas import tpu as pltpu
from jax.experimental.pallas import tpu_sc as plsc

N = 320000
D = 128
S = 1024

NC = 2
NS = 16
NW = NC * NS
ROWS_PER_TILE = N // NW
CHUNK = 400
NCHUNKS = ROWS_PER_TILE // CHUNK
SUB = 80
NSUB = CHUNK // SUB
SEG_PER_TILE = S // NS


def _sc_segment_sum(x, idx, zsums, zcounts, ones):
    mesh = plsc.VectorSubcoreMesh(core_axis_name="c", subcore_axis_name="s")

    @functools.partial(
        pl.kernel,
        out_type=[
            jax.ShapeDtypeStruct((NC, S, D), jnp.float32),
            jax.ShapeDtypeStruct((NC * S,), jnp.float32),
        ],
        mesh=mesh,
        scratch_types=[
            [pltpu.VMEM((CHUNK, D), jnp.float32) for _ in range(2)],
            [[pltpu.VMEM((SUB,), jnp.int32) for _ in range(NSUB)]
             for _ in range(2)],
            pltpu.VMEM((SUB,), jnp.float32),
            pltpu.VMEM((SEG_PER_TILE,), jnp.float32),
            pltpu.VMEM_SHARED((S, D), jnp.float32),
            pltpu.VMEM_SHARED((S,), jnp.float32),
            [pltpu.SemaphoreType.DMA for _ in range(2)],
        ],
    )
    def seg_sum(x_hbm, idx_hbm, zs_hbm, zc_hbm, ones_hbm, out_sums, out_counts,
                xbufs, idxbufs, onesbuf, cbuf, acc_sums, acc_counts, sems):
        cid = lax.axis_index("c")
        sid = lax.axis_index("s")
        wid = cid * NS + sid
        seg0 = sid * SEG_PER_TILE
        row0 = wid * ROWS_PER_TILE

        pltpu.sync_copy(zs_hbm.at[pl.ds(seg0, SEG_PER_TILE)],
                        acc_sums.at[pl.ds(seg0, SEG_PER_TILE)])
        pltpu.sync_copy(zc_hbm.at[pl.ds(seg0, SEG_PER_TILE)], cbuf)
        pltpu.sync_copy(cbuf, acc_counts.at[pl.ds(seg0, SEG_PER_TILE)])
        pltpu.sync_copy(ones_hbm, onesbuf)
        plsc.subcore_barrier()

        def issue_loads(ci, b):
            base = row0 + ci * CHUNK
            pltpu.async_copy(x_hbm.at[pl.ds(base, CHUNK)], xbufs[b], sems[b])
            for j in range(NSUB):
                pltpu.async_copy(idx_hbm.at[pl.ds(base + j * SUB, SUB)],
                                 idxbufs[b][j], sems[b])

        def drain_loads(b):
            pltpu.make_async_copy(x_hbm.at[pl.ds(0, CHUNK)], xbufs[b],
                                  sems[b]).wait()
            for j in range(NSUB):
                pltpu.make_async_copy(idx_hbm.at[pl.ds(0, SUB)],
                                      idxbufs[b][j], sems[b]).wait()

        def scatter(b):
            for j in range(NSUB):
                pltpu.sync_copy(xbufs[b].at[pl.ds(j * SUB, SUB)],
                                acc_sums.at[idxbufs[b][j]], add=True)
                pltpu.sync_copy(onesbuf, acc_counts.at[idxbufs[b][j]],
                                add=True)

        issue_loads(0, 0)

        def body(i, carry):
            ci = i * 2
            drain_loads(0)
            issue_loads(ci + 1, 1)
            scatter(0)
            drain_loads(1)
            issue_loads(ci + 2, 0)
            scatter(1)
            return carry

        lax.fori_loop(0, (NCHUNKS - 1) // 2, body, 0)
        drain_loads(0)
        scatter(0)
        plsc.subcore_barrier()

        pltpu.sync_copy(acc_sums.at[pl.ds(seg0, SEG_PER_TILE)],
                        out_sums.at[cid, pl.ds(seg0, SEG_PER_TILE)])
        pltpu.sync_copy(acc_counts.at[pl.ds(seg0, SEG_PER_TILE)], cbuf)
        pltpu.sync_copy(cbuf, out_counts.at[pl.ds(cid * S + seg0, SEG_PER_TILE)])

    return seg_sum(x, idx, zsums, zcounts, ones)


def _finalize(psums, pcounts_t, gamma1, beta1, W1, b1, W2, b2, gamma2, beta2):
    def body(ps_ref, pc_ref, g1_ref, be1_ref, w1_ref, b1_ref, w2_ref, b2_ref,
             g2_ref, be2_ref, out_ref):
        sums = ps_ref[0] + ps_ref[1]
        counts = jnp.sum(pc_ref[...], axis=1, keepdims=True)
        counts = jnp.maximum(counts, 1.0)
        h = sums / counts

        mean = jnp.mean(h, axis=-1, keepdims=True)
        var = jnp.mean((h - mean) * (h - mean), axis=-1, keepdims=True)
        h = (h - mean) * lax.rsqrt(var + 1e-5) * g1_ref[...] + be1_ref[...]

        y = lax.dot_general(h, w1_ref[...], (((1,), (1,)), ((), ())),
                            preferred_element_type=jnp.float32) + b1_ref[...]
        y = jnp.where(y > 0.0, y, jnp.exp(jnp.minimum(y, 0.0)) - 1.0)
        y = lax.dot_general(y, w2_ref[...], (((1,), (1,)), ((), ())),
                            preferred_element_type=jnp.float32) + b2_ref[...]
        y = y + h

        mean2 = jnp.mean(y, axis=-1, keepdims=True)
        var2 = jnp.mean((y - mean2) * (y - mean2), axis=-1, keepdims=True)
        out_ref[...] = ((y - mean2) * lax.rsqrt(var2 + 1e-5) * g2_ref[...]
                        + be2_ref[...])

    return pl.pallas_call(
        body,
        out_shape=jax.ShapeDtypeStruct((S, D), jnp.float32),
    )(psums, pcounts_t, gamma1.reshape(1, D), beta1.reshape(1, D), W1,
      b1.reshape(1, D), W2, b2.reshape(1, D), gamma2.reshape(1, D),
      beta2.reshape(1, D))


def kernel(x, graph_index, gamma1, beta1, W1, b1, W2, b2, gamma2, beta2):
    idx = graph_index.astype(jnp.int32)
    zsums = jnp.zeros((S, D), jnp.float32)
    zcounts = jnp.zeros((S,), jnp.float32)
    ones = jnp.ones((SUB,), jnp.float32)
    psums, pcounts = _sc_segment_sum(x, idx, zsums, zcounts, ones)
    pcounts_t = pcounts.reshape(NC, S).T
    return _finalize(psums, pcounts_t, gamma1, beta1, W1, b1, W2, b2, gamma2,
                     beta2)


# speedup vs baseline: 7.4545x; 1.0122x over previous
import functools

import jax
import jax.numpy as jnp
from jax import lax
from jax.experimental import pallas as pl
from jax.experimental.pallas import tpu as pltpu
from jax.experimental.pallas import tpu_sc as plsc

N = 320000
D = 128
S = 1024

NC = 2
NS = 16
NW = NC * NS
ROWS_PER_TILE = N // NW
CHUNK = 400
NCHUNKS = ROWS_PER_TILE // CHUNK
SUB = 80
NSUB = CHUNK // SUB
SEG_PER_TILE = S // NS


def _sc_segment_sum(x, idx, zsums, zcounts, ones):
    mesh = plsc.VectorSubcoreMesh(core_axis_name="c", subcore_axis_name="s")

    @functools.partial(
        pl.kernel,
        out_type=[
            jax.ShapeDtypeStruct((NC, S, D), jnp.float32),
            jax.ShapeDtypeStruct((NC * S,), jnp.float32),
        ],
        mesh=mesh,
        scratch_types=[
            [pltpu.VMEM((CHUNK, D), jnp.float32) for _ in range(2)],
            [[pltpu.VMEM((SUB,), jnp.int32) for _ in range(NSUB)]
             for _ in range(2)],
            pltpu.VMEM((SUB,), jnp.float32),
            pltpu.VMEM((SEG_PER_TILE,), jnp.float32),
            pltpu.VMEM_SHARED((S, D), jnp.float32),
            pltpu.VMEM_SHARED((S,), jnp.float32),
            [pltpu.SemaphoreType.DMA for _ in range(2)],
            [pltpu.SemaphoreType.DMA for _ in range(2)],
        ],
    )
    def seg_sum(x_hbm, idx_hbm, zs_hbm, zc_hbm, ones_hbm, out_sums, out_counts,
                xbufs, idxbufs, onesbuf, cbuf, acc_sums, acc_counts, sems,
                ssems):
        cid = lax.axis_index("c")
        sid = lax.axis_index("s")
        wid = cid * NS + sid
        seg0 = sid * SEG_PER_TILE
        row0 = wid * ROWS_PER_TILE

        pltpu.sync_copy(zs_hbm.at[pl.ds(seg0, SEG_PER_TILE)],
                        acc_sums.at[pl.ds(seg0, SEG_PER_TILE)])
        pltpu.sync_copy(zc_hbm.at[pl.ds(seg0, SEG_PER_TILE)], cbuf)
        pltpu.sync_copy(cbuf, acc_counts.at[pl.ds(seg0, SEG_PER_TILE)])
        pltpu.sync_copy(ones_hbm, onesbuf)
        plsc.subcore_barrier()

        def issue_loads(ci, b):
            base = row0 + ci * CHUNK
            pltpu.async_copy(x_hbm.at[pl.ds(base, CHUNK)], xbufs[b], sems[b])
            for j in range(NSUB):
                pltpu.async_copy(idx_hbm.at[pl.ds(base + j * SUB, SUB)],
                                 idxbufs[b][j], sems[b])

        def drain_loads(b):
            pltpu.make_async_copy(x_hbm.at[pl.ds(0, CHUNK)], xbufs[b],
                                  sems[b]).wait()
            for j in range(NSUB):
                pltpu.make_async_copy(idx_hbm.at[pl.ds(0, SUB)],
                                      idxbufs[b][j], sems[b]).wait()

        def issue_scatters(b):
            for j in range(NSUB):
                pltpu.async_copy(xbufs[b].at[pl.ds(j * SUB, SUB)],
                                 acc_sums.at[idxbufs[b][j]], ssems[b],
                                 add=True)
                pltpu.async_copy(onesbuf, acc_counts.at[idxbufs[b][j]],
                                 ssems[b], add=True)

        def drain_scatters(b):
            for j in range(NSUB):
                pltpu.make_async_copy(xbufs[b].at[pl.ds(j * SUB, SUB)],
                                      acc_sums.at[idxbufs[b][j]],
                                      ssems[b]).wait()
                pltpu.make_async_copy(onesbuf, acc_counts.at[idxbufs[b][j]],
                                      ssems[b]).wait()

        issue_loads(0, 0)

        def body(i, carry):
            ci = i * 2
            drain_loads(0)
            issue_loads(ci + 1, 1)
            issue_scatters(0)
            drain_loads(1)
            drain_scatters(0)
            issue_loads(ci + 2, 0)
            issue_scatters(1)
            drain_scatters(1)
            return carry

        lax.fori_loop(0, (NCHUNKS - 1) // 2, body, 0)
        drain_loads(0)
        issue_scatters(0)
        drain_scatters(0)
        plsc.subcore_barrier()

        pltpu.sync_copy(acc_sums.at[pl.ds(seg0, SEG_PER_TILE)],
                        out_sums.at[cid, pl.ds(seg0, SEG_PER_TILE)])
        pltpu.sync_copy(acc_counts.at[pl.ds(seg0, SEG_PER_TILE)], cbuf)
        pltpu.sync_copy(cbuf, out_counts.at[pl.ds(cid * S + seg0, SEG_PER_TILE)])

    return seg_sum(x, idx, zsums, zcounts, ones)


def _finalize(psums, pcounts_t, gamma1, beta1, W1, b1, W2, b2, gamma2, beta2):
    def body(ps_ref, pc_ref, g1_ref, be1_ref, w1_ref, b1_ref, w2_ref, b2_ref,
             g2_ref, be2_ref, out_ref):
        sums = ps_ref[0] + ps_ref[1]
        counts = jnp.sum(pc_ref[...], axis=1, keepdims=True)
        counts = jnp.maximum(counts, 1.0)
        h = sums / counts

        mean = jnp.mean(h, axis=-1, keepdims=True)
        var = jnp.mean((h - mean) * (h - mean), axis=-1, keepdims=True)
        h = (h - mean) * lax.rsqrt(var + 1e-5) * g1_ref[...] + be1_ref[...]

        y = lax.dot_general(h, w1_ref[...], (((1,), (1,)), ((), ())),
                            preferred_element_type=jnp.float32) + b1_ref[...]
        y = jnp.where(y > 0.0, y, jnp.exp(jnp.minimum(y, 0.0)) - 1.0)
        y = lax.dot_general(y, w2_ref[...], (((1,), (1,)), ((), ())),
                            preferred_element_type=jnp.float32) + b2_ref[...]
        y = y + h

        mean2 = jnp.mean(y, axis=-1, keepdims=True)
        var2 = jnp.mean((y - mean2) * (y - mean2), axis=-1, keepdims=True)
        out_ref[...] = ((y - mean2) * lax.rsqrt(var2 + 1e-5) * g2_ref[...]
                        + be2_ref[...])

    return pl.pallas_call(
        body,
        out_shape=jax.ShapeDtypeStruct((S, D), jnp.float32),
    )(psums, pcounts_t, gamma1.reshape(1, D), beta1.reshape(1, D), W1,
      b1.reshape(1, D), W2, b2.reshape(1, D), gamma2.reshape(1, D),
      beta2.reshape(1, D))


def kernel(x, graph_index, gamma1, beta1, W1, b1, W2, b2, gamma2, beta2):
    idx = graph_index.astype(jnp.int32)
    zsums = jnp.zeros((S, D), jnp.float32)
    zcounts = jnp.zeros((S,), jnp.float32)
    ones = jnp.ones((SUB,), jnp.float32)
    psums, pcounts = _sc_segment_sum(x, idx, zsums, zcounts, ones)
    pcounts_t = pcounts.reshape(NC, S).T
    return _finalize(psums, pcounts_t, gamma1, beta1, W1, b1, W2, b2, gamma2,
                     beta2)
